# baseline (speedup 1.0000x reference)
import functools

import jax
import jax.numpy as jnp
from jax.experimental import pallas as pl
from jax.experimental.pallas import tpu as pltpu
from jax.experimental.pallas import tpu_sc as plsc


def _logits_kernel(ts_ref, wr_ref, br_ref, logits_ref):
    pooled = jnp.mean(ts_ref[...], axis=1)
    logits = jnp.dot(pooled, wr_ref[...],
                     preferred_element_type=jnp.float32) + br_ref[...]
    bsz = logits.shape[0]
    pad = jnp.full((bsz, 8), -1e30, jnp.float32)
    logits_ref[...] = jnp.concatenate([logits, pad], axis=1)


def _sc_router_kernel(logits_hbm, probs_hbm, ti_hbm, tv_hbm, aux_hbm,
                      logits_v, probs_v, ti_v, tv_v, aux_v):
    @pl.when(jnp.logical_and(jax.lax.axis_index("c") == 0,
                             jax.lax.axis_index("s") == 0))
    def _():
        pltpu.sync_copy(logits_hbm, logits_v)
        iota = jax.lax.iota(jnp.int32, 16)
        me = jnp.zeros((16,), jnp.float32)
        for r in range(4):
            v = logits_v[r]
            m = jnp.max(v, axis=0)
            e = jnp.exp(v - m)
            p = e / jnp.sum(e, axis=0)
            probs_v[r] = p
            sk, sv = plsc.sort_key_val(p, iota, descending=True)
            s12 = jnp.sum(jnp.where(iota < 2, sk, 0.0), axis=0)
            tv_v[r] = sk / s12
            ti_v[r] = sv
            me = me + p
        me = me * 0.25
        aux = jnp.sum(me * me, axis=0) * 8.0
        aux_v[...] = jnp.full((16,), aux, jnp.float32)
        pltpu.sync_copy(probs_v, probs_hbm)
        pltpu.sync_copy(ti_v, ti_hbm)
        pltpu.sync_copy(tv_v, tv_hbm)
        pltpu.sync_copy(aux_v, aux_hbm)


def _moe_kernel(nf, idx_ref, wv_ref, x_ref, w1_ref, b1_ref, w2_ref, b2_ref,
                g_ref, bt_ref, out_ref):
    i_bk = pl.program_id(0)
    i_f = pl.program_id(1)
    w = wv_ref[i_bk]

    @pl.when(jnp.logical_and(i_bk % 2 == 0, i_f == 0))
    def _init():
        out_ref[0] = jnp.zeros_like(out_ref[0])

    t = jnp.dot(x_ref[0], w1_ref[0],
                preferred_element_type=jnp.float32).astype(jnp.bfloat16)
    t = t + b1_ref[0].astype(jnp.bfloat16)
    c1 = jnp.bfloat16(0.7978845608028654)
    c2 = jnp.bfloat16(0.7978845608028654 * 0.044715)
    t2 = t * t
    u = t * (c1 + c2 * t2)
    th = jnp.tanh(u)
    half = jnp.bfloat16(0.5)
    h = t * (half + half * th)
    part = jnp.dot(h, (w * w2_ref[0]).astype(jnp.bfloat16),
                   preferred_element_type=jnp.float32)

    @pl.when(i_f == 0)
    def _bias():
        out_ref[0] += w * b2_ref[0]

    out_ref[0] += part

    @pl.when(jnp.logical_and(i_bk % 2 == 1, i_f == nf - 1))
    def _epilogue():
        mixed = out_ref[0]
        mu = jnp.mean(mixed, axis=-1, keepdims=True)
        var = jnp.mean((mixed - mu) ** 2, axis=-1, keepdims=True)
        normed = (mixed - mu) * jax.lax.rsqrt(var + 1e-5)
        out_ref[0] = x_ref[0] + normed * g_ref[...] + bt_ref[...]


def kernel(x, text_state, W1, b1, W2, b2, Wr, br, gamma, beta):
    b, s, d = x.shape
    E, _, ff = W1.shape
    k = 2

    logits_pad = pl.pallas_call(
        _logits_kernel,
        out_shape=jax.ShapeDtypeStruct((b, 16), jnp.float32),
    )(text_state, Wr, br.reshape(1, E))

    sc_router = pl.kernel(
        _sc_router_kernel,
        out_type=(
            jax.ShapeDtypeStruct((b, 16), jnp.float32),
            jax.ShapeDtypeStruct((b, 16), jnp.int32),
            jax.ShapeDtypeStruct((b, 16), jnp.float32),
            jax.ShapeDtypeStruct((16,), jnp.float32),
        ),
        mesh=plsc.VectorSubcoreMesh(core_axis_name="c", subcore_axis_name="s"),
        compiler_params=pltpu.CompilerParams(needs_layout_passes=False),
        scratch_types=[
            pltpu.VMEM((b, 16), jnp.float32),
            pltpu.VMEM((b, 16), jnp.float32),
            pltpu.VMEM((b, 16), jnp.int32),
            pltpu.VMEM((b, 16), jnp.float32),
            pltpu.VMEM((16,), jnp.float32),
        ],
    )
    probs16, ti16, tv16, aux16 = sc_router(logits_pad)
    probs = probs16[:, :E]
    ti = ti16[:, :k]
    tv = tv16[:, :k]
    aux = aux16[0]

    ft = 1024
    nf = ff // ft
    b1r = b1.reshape(E, 1, ff)
    b2r = b2.reshape(E, 1, d)

    grid_spec = pltpu.PrefetchScalarGridSpec(
        num_scalar_prefetch=2,
        grid=(k * b, nf),
        in_specs=[
            pl.BlockSpec((1, s, d), lambda i, j, idx, wv: (i // 2, 0, 0),
                         pipeline_mode=pl.Buffered(buffer_count=1)),
            pl.BlockSpec((1, d, ft), lambda i, j, idx, wv: (idx[i], 0, j)),
            pl.BlockSpec((1, 1, ft), lambda i, j, idx, wv: (idx[i], 0, j)),
            pl.BlockSpec((1, ft, d), lambda i, j, idx, wv: (idx[i], j, 0)),
            pl.BlockSpec((1, 1, d), lambda i, j, idx, wv: (idx[i], 0, 0)),
            pl.BlockSpec((1, d), lambda i, j, idx, wv: (0, 0)),
            pl.BlockSpec((1, d), lambda i, j, idx, wv: (0, 0)),
        ],
        out_specs=pl.BlockSpec((1, s, d), lambda i, j, idx, wv: (i // 2, 0, 0),
                               pipeline_mode=pl.Buffered(buffer_count=1)),
    )

    out = pl.pallas_call(
        functools.partial(_moe_kernel, nf),
        grid_spec=grid_spec,
        out_shape=jax.ShapeDtypeStruct((b, s, d), jnp.float32),
        compiler_params=pltpu.CompilerParams(
            vmem_limit_bytes=63 * 1024 * 1024),
    )(ti.reshape(k * b), tv.reshape(k * b), x, W1, b1r, W2, b2r,
      gamma.reshape(1, d), beta.reshape(1, d))

    return out, probs, aux.reshape(())
